# baseline (speedup 1.0000x reference)
import functools
import jax
import jax.numpy as jnp
from jax import lax
from jax.experimental import pallas as pl
from jax.experimental.pallas import tpu as pltpu
from jax.experimental.pallas import tpu_sc as plsc

_D = 64
_DP = 128
_SCALE = 8.0
_NC = 2
_NS = 16
_NW = _NC * _NS
_CHUNK = 128
_LANES = 16
_NB = 5
_TBLK = 32768


def _row_major_table(table_t, vocab):
  def body(tt_ref, out_ref):
    x = tt_ref[...]
    eye = jnp.eye(_D, dtype=jnp.float32) * _SCALE
    y = jax.lax.dot_general(
        x, eye, dimension_numbers=(((0,), (0,)), ((), ())),
        preferred_element_type=jnp.float32)
    out_ref[:, 0:_D] = y

  return pl.pallas_call(
      body,
      grid=((vocab + _TBLK - 1) // _TBLK,),
      in_specs=[pl.BlockSpec((_D, _TBLK), lambda j: (0, j))],
      out_specs=pl.BlockSpec((_TBLK, _DP), lambda j: (j, 0)),
      out_shape=jax.ShapeDtypeStruct((vocab, _DP), jnp.float32),
  )(table_t)


def _make_emb_kernel(n_chunks: int, batch: int, seq: int, vocab: int):
  b_per_w = n_chunks * _CHUNK
  n_groups = n_chunks // _NB
  nbt = batch // _CHUNK
  mesh = plsc.VectorSubcoreMesh(core_axis_name="c", subcore_axis_name="s",
                                num_cores=_NC, num_subcores=_NS)

  @functools.partial(
      pl.kernel,
      mesh=mesh,
      compiler_params=pltpu.CompilerParams(use_tc_tiling_on_sc=False,
                                           needs_layout_passes=False),
      out_type=jax.ShapeDtypeStruct((seq, _D // 8, nbt, 8, _CHUNK),
                                    jnp.float32),
      scratch_types=[
          pltpu.VMEM((n_chunks, _CHUNK), jnp.int32),
          [pltpu.VMEM((_CHUNK, _DP), jnp.float32) for _ in range(_NB)],
          [pltpu.VMEM((_D // 8, 8, _CHUNK), jnp.float32) for _ in range(2)],
          [pltpu.SemaphoreType.DMA for _ in range(_NB)],
          [pltpu.SemaphoreType.DMA for _ in range(2)],
      ],
  )
  def emb(tokens_hbm, table_hbm, out_hbm, idx_v, rows, wbuf, gsem, wsem):
    wid = lax.axis_index("s") * _NC + lax.axis_index("c")
    base = wid * b_per_w
    pltpu.sync_copy(tokens_hbm.at[wid], idx_v)
    iot = jax.lax.iota(jnp.int32, _LANES)

    def out_dst(j):
      f = base + j * _CHUNK
      s = f // batch
      bt = (f // _CHUNK) % nbt
      return out_hbm.at[s, :, bt]

    perms = [lax.rem(iot + k, jnp.int32(_LANES)) for k in range(_LANES)]
    perms_hi = [lax.shift_right_logical(p, 3) for p in perms]
    perms_lo = [lax.bitwise_and(p, jnp.int32(7)) for p in perms]

    def transform_chunk(j, b, bw):
      def blk_body(m, c2):
        db = m // 8
        gb = m % 8
        rowi = iot + gb * _LANES
        d0 = db * _LANES
        dt0 = jnp.full((_LANES,), db * 2, jnp.int32)
        for k in range(_LANES):
          v = plsc.load_gather(rows[b], [rowi, perms[k] + d0])
          plsc.store_scatter(wbuf[bw], [dt0 + perms_hi[k], perms_lo[k], rowi],
                             v)
        return c2

      lax.fori_loop(0, (_D // _LANES) * (_CHUNK // _LANES), blk_body, 0)

    def do_chunk(j, b, bw, first, last):
      pltpu.make_async_copy(table_hbm.at[idx_v.at[j]], rows[b],
                            gsem[b]).wait()
      if not first:
        pltpu.make_async_copy(wbuf[bw], out_dst(j), wsem[bw]).wait()
      transform_chunk(j, b, bw)
      if not last:
        pltpu.async_copy(table_hbm.at[idx_v.at[j + _NB]], rows[b], gsem[b])
      pltpu.async_copy(wbuf[bw], out_dst(j), wsem[bw])

    span = 2 * _NB
    n_spans = n_chunks // span

    for b in range(_NB):
      pltpu.async_copy(table_hbm.at[idx_v.at[jnp.int32(b)]], rows[b], gsem[b])

    for b in range(span):
      do_chunk(jnp.int32(b), b % _NB, b % 2, first=(b < 2), last=False)

    def group_body(g, carry):
      for b in range(span):
        do_chunk(g * span + b, b % _NB, b % 2, first=False, last=False)
      return carry

    lax.fori_loop(1, n_spans - 1, group_body, 0)

    for b in range(span):
      do_chunk(jnp.int32((n_spans - 1) * span + b), b % _NB, b % 2,
               first=False, last=(b >= span - _NB))

    for bw in range(2):
      pltpu.make_async_copy(
          wbuf[bw], out_dst(jnp.int32(n_chunks - 2 + bw)), wsem[bw]).wait()

  return emb


def kernel(tokens, table):
  batch, seq = tokens.shape
  vocab = table.shape[0]
  total = batch * seq
  n_chunks = total // (_NW * _CHUNK)
  tokens_t = tokens.T.reshape(_NW, n_chunks, _CHUNK).astype(jnp.int32)
  table_p = _row_major_table(table.T, vocab)
  out5 = _make_emb_kernel(n_chunks, batch, seq, vocab)(tokens_t, table_p)
  return out5.transpose(2, 4, 0, 1, 3).reshape(batch, seq, _D)


# speedup vs baseline: 1.0038x; 1.0038x over previous
import functools
import jax
import jax.numpy as jnp
from jax import lax
from jax.experimental import pallas as pl
from jax.experimental.pallas import tpu as pltpu
from jax.experimental.pallas import tpu_sc as plsc

_D = 64
_DP = 128
_SCALE = 8.0
_NC = 2
_NS = 16
_NW = _NC * _NS
_CHUNK = 128
_LANES = 16
_NB = 4
_TBLK = 32768


def _row_major_table(table_t, vocab):
  def body(tt_ref, out_ref):
    x = tt_ref[...]
    eye = jnp.eye(_D, dtype=jnp.float32) * _SCALE
    y = jax.lax.dot_general(
        x, eye, dimension_numbers=(((0,), (0,)), ((), ())),
        preferred_element_type=jnp.float32)
    out_ref[:, 0:_D] = y

  return pl.pallas_call(
      body,
      grid=((vocab + _TBLK - 1) // _TBLK,),
      in_specs=[pl.BlockSpec((_D, _TBLK), lambda j: (0, j))],
      out_specs=pl.BlockSpec((_TBLK, _DP), lambda j: (j, 0)),
      out_shape=jax.ShapeDtypeStruct((vocab, _DP), jnp.float32),
  )(table_t)


def _make_emb_kernel(n_chunks: int, batch: int, seq: int, vocab: int):
  b_per_w = n_chunks * _CHUNK
  n_groups = n_chunks // _NB
  nbt = batch // _CHUNK
  mesh = plsc.VectorSubcoreMesh(core_axis_name="c", subcore_axis_name="s",
                                num_cores=_NC, num_subcores=_NS)

  @functools.partial(
      pl.kernel,
      mesh=mesh,
      compiler_params=pltpu.CompilerParams(use_tc_tiling_on_sc=False,
                                           needs_layout_passes=False),
      out_type=jax.ShapeDtypeStruct((seq, _D // 8, nbt, 8, _CHUNK),
                                    jnp.float32),
      scratch_types=[
          pltpu.VMEM((n_chunks, _CHUNK), jnp.int32),
          [pltpu.VMEM((_CHUNK, _DP), jnp.float32) for _ in range(_NB)],
          [pltpu.VMEM((_D // 8, 8, _CHUNK), jnp.float32) for _ in range(2)],
          [pltpu.SemaphoreType.DMA for _ in range(_NB)],
          [pltpu.SemaphoreType.DMA for _ in range(2)],
      ],
  )
  def emb(tokens_hbm, table_hbm, out_hbm, idx_v, rows, wbuf, gsem, wsem):
    wid = lax.axis_index("s") * _NC + lax.axis_index("c")
    base = wid * b_per_w
    pltpu.sync_copy(tokens_hbm.at[wid], idx_v)
    iot = jax.lax.iota(jnp.int32, _LANES)

    def out_dst(j):
      f = base + j * _CHUNK
      s = f // batch
      bt = (f // _CHUNK) % nbt
      return out_hbm.at[s, :, bt]

    perms = [lax.rem(iot + k, jnp.int32(_LANES)) for k in range(_LANES)]
    perms_hi = [lax.shift_right_logical(p, 3) for p in perms]
    perms_lo = [lax.bitwise_and(p, jnp.int32(7)) for p in perms]

    def transform_chunk(j, b, bw):
      def blk_body(m, c2):
        db = m // 8
        gb = m % 8
        rowi = iot + gb * _LANES
        d0 = db * _LANES
        dt0 = jnp.full((_LANES,), db * 2, jnp.int32)
        for k in range(_LANES):
          v = plsc.load_gather(rows[b], [rowi, perms[k] + d0])
          plsc.store_scatter(wbuf[bw], [dt0 + perms_hi[k], perms_lo[k], rowi],
                             v)
        return c2

      lax.fori_loop(0, (_D // _LANES) * (_CHUNK // _LANES), blk_body, 0)

    def do_chunk(j, b, bw, first, last):
      pltpu.make_async_copy(table_hbm.at[idx_v.at[j]], rows[b],
                            gsem[b]).wait()
      if not first:
        pltpu.make_async_copy(wbuf[bw], out_dst(j), wsem[bw]).wait()
      transform_chunk(j, b, bw)
      if not last:
        pltpu.async_copy(table_hbm.at[idx_v.at[j + _NB]], rows[b], gsem[b])
      pltpu.async_copy(wbuf[bw], out_dst(j), wsem[bw])

    span = 2 * _NB
    n_spans = n_chunks // span

    for b in range(_NB):
      pltpu.async_copy(table_hbm.at[idx_v.at[jnp.int32(b)]], rows[b], gsem[b])

    for b in range(span):
      do_chunk(jnp.int32(b), b % _NB, b % 2, first=(b < 2), last=False)

    def group_body(g, carry):
      for b in range(span):
        do_chunk(g * span + b, b % _NB, b % 2, first=False, last=False)
      return carry

    lax.fori_loop(1, n_spans - 1, group_body, 0)

    for b in range(span):
      do_chunk(jnp.int32((n_spans - 1) * span + b), b % _NB, b % 2,
               first=False, last=(b >= span - _NB))

    for bw in range(2):
      pltpu.make_async_copy(
          wbuf[bw], out_dst(jnp.int32(n_chunks - 2 + bw)), wsem[bw]).wait()

  return emb


def kernel(tokens, table):
  batch, seq = tokens.shape
  vocab = table.shape[0]
  total = batch * seq
  n_chunks = total // (_NW * _CHUNK)
  tokens_t = tokens.T.reshape(_NW, n_chunks, _CHUNK).astype(jnp.int32)
  table_p = _row_major_table(table.T, vocab)
  out5 = _make_emb_kernel(n_chunks, batch, seq, vocab)(tokens_t, table_p)
  return out5.transpose(2, 4, 0, 1, 3).reshape(batch, seq, _D)


# speedup vs baseline: 1.0159x; 1.0121x over previous
import functools
import jax
import jax.numpy as jnp
from jax import lax
from jax.experimental import pallas as pl
from jax.experimental.pallas import tpu as pltpu
from jax.experimental.pallas import tpu_sc as plsc

_D = 64
_DP = 128
_SCALE = 8.0
_NC = 2
_NS = 16
_NW = _NC * _NS
_CHUNK = 128
_LANES = 16
_NB = 4
_TBLK = 32768


def _row_major_table(table_t, vocab):
  def body(tt_ref, out_ref):
    x = tt_ref[...]
    eye = jnp.eye(_D, dtype=jnp.float32) * _SCALE
    y = jax.lax.dot_general(
        x, eye, dimension_numbers=(((0,), (0,)), ((), ())),
        preferred_element_type=jnp.float32)
    out_ref[:, 0:_D] = y

  return pl.pallas_call(
      body,
      grid=((vocab + _TBLK - 1) // _TBLK,),
      in_specs=[pl.BlockSpec((_D, _TBLK), lambda j: (0, j))],
      out_specs=pl.BlockSpec((_TBLK, _DP), lambda j: (j, 0)),
      out_shape=jax.ShapeDtypeStruct((vocab, _DP), jnp.float32),
  )(table_t)


def _make_emb_kernel(n_chunks: int, batch: int, seq: int, vocab: int):
  b_per_w = n_chunks * _CHUNK
  n_groups = n_chunks // _NB
  nbt = batch // _CHUNK
  mesh = plsc.VectorSubcoreMesh(core_axis_name="c", subcore_axis_name="s",
                                num_cores=_NC, num_subcores=_NS)

  @functools.partial(
      pl.kernel,
      mesh=mesh,
      compiler_params=pltpu.CompilerParams(use_tc_tiling_on_sc=False,
                                           needs_layout_passes=False),
      out_type=jax.ShapeDtypeStruct((seq, _D // 8, nbt, 8, _CHUNK),
                                    jnp.float32),
      scratch_types=[
          pltpu.VMEM((n_chunks, _CHUNK), jnp.int32),
          [pltpu.VMEM((_CHUNK, _DP), jnp.float32) for _ in range(_NB)],
          [pltpu.VMEM((_D // 8, 8, _CHUNK), jnp.float32) for _ in range(2)],
          [pltpu.SemaphoreType.DMA for _ in range(_NB)],
          [pltpu.SemaphoreType.DMA for _ in range(2)],
      ],
  )
  def emb(tokens_hbm, table_hbm, out_hbm, idx_v, rows, wbuf, gsem, wsem):
    wid = lax.axis_index("s") * _NC + lax.axis_index("c")
    base = wid * b_per_w
    pltpu.sync_copy(tokens_hbm.at[wid], idx_v)
    iot = jax.lax.iota(jnp.int32, _LANES)

    def out_dst(j):
      f = base + j * _CHUNK
      s = f // batch
      bt = (f // _CHUNK) % nbt
      return out_hbm.at[s, :, bt]

    perms = [lax.rem(iot + k, jnp.int32(_LANES)) for k in range(_LANES)]
    perms_hi = [lax.shift_right_logical(p, 3) for p in perms]
    perms_lo = [lax.bitwise_and(p, jnp.int32(7)) for p in perms]

    def transform_chunk(j, b, bw):
      def blk_body(m, c2):
        db = m // 8
        gb = m % 8
        rowi = iot + gb * _LANES
        d0 = db * _LANES
        dt0 = jnp.full((_LANES,), db * 2, jnp.int32)
        for k in range(_LANES):
          v = plsc.load_gather(rows[b], [rowi, perms[k] + d0])
          plsc.store_scatter(wbuf[bw], [dt0 + perms_hi[k], perms_lo[k], rowi],
                             v)
        return c2

      lax.fori_loop(0, (_D // _LANES) * (_CHUNK // _LANES), blk_body, 0)

    def do_chunk(j, b, first, last):
      bw = b % 2
      pltpu.make_async_copy(table_hbm.at[idx_v.at[j]], rows[b],
                            gsem[b]).wait()
      if not first:
        pltpu.make_async_copy(wbuf[bw], out_dst(j), wsem[bw]).wait()
      transform_chunk(j, b, bw)
      if not last:
        pltpu.async_copy(table_hbm.at[idx_v.at[j + _NB]], rows[b], gsem[b])
      pltpu.async_copy(wbuf[bw], out_dst(j), wsem[bw])

    for b in range(_NB):
      pltpu.async_copy(table_hbm.at[idx_v.at[jnp.int32(b)]], rows[b], gsem[b])

    for b in range(_NB):
      do_chunk(jnp.int32(b), b, first=(b < 2), last=False)

    def group_body(g, carry):
      for b in range(_NB):
        do_chunk(g * _NB + b, b, first=False, last=False)
      return carry

    lax.fori_loop(1, n_groups - 1, group_body, 0)

    for b in range(_NB):
      do_chunk(jnp.int32((n_groups - 1) * _NB + b), b, first=False, last=True)

    for bw in range(2):
      pltpu.make_async_copy(
          wbuf[bw], out_dst(jnp.int32(n_chunks - 2 + bw)), wsem[bw]).wait()

  return emb


def kernel(tokens, table):
  batch, seq = tokens.shape
  vocab = table.shape[0]
  total = batch * seq
  n_chunks = total // (_NW * _CHUNK)
  tokens_t = tokens.T.reshape(_NW, n_chunks, _CHUNK).astype(jnp.int32)
  table_p = _row_major_table(table.T, vocab)
  out5 = _make_emb_kernel(n_chunks, batch, seq, vocab)(tokens_t, table_p)
  return out5.transpose(2, 4, 0, 1, 3).reshape(batch, seq, _D)
